# baseline (speedup 1.0000x reference)
import functools

import jax
import jax.numpy as jnp
from jax import lax
from jax.experimental import pallas as pl
from jax.experimental.pallas import tpu as pltpu
from jax.experimental.pallas import tpu_sc as plsc

NUM_CORES = 2
NUM_SUBCORES = 16
NW = NUM_CORES * NUM_SUBCORES

CHUNK = 128
LANE = 16


@functools.partial(jax.jit, static_argnames=("v", "d", "dp"))
def _gather_t(idx, table_pad, *, v, d, dp):
    nw, n_l, _ = idx.shape
    batch = nw * CHUNK
    mesh = plsc.VectorSubcoreMesh(core_axis_name="c", subcore_axis_name="s")

    @functools.partial(
        pl.kernel,
        out_type=jax.ShapeDtypeStruct((n_l * d, batch), jnp.float32),
        mesh=mesh,
        scratch_types=[
            pltpu.VMEM((n_l, CHUNK), jnp.int32),
            pltpu.VMEM((2, CHUNK, dp), jnp.float32),
            pltpu.VMEM((d, CHUNK), jnp.float32),
            pltpu.SemaphoreType.DMA((2,)),
        ],
        compiler_params=pltpu.CompilerParams(use_tc_tiling_on_sc=False, needs_layout_passes=False),
    )
    def run(idx_hbm, table_hbm, out_hbm, idx_v, bufs, buf_t, gsem):
        wid = lax.axis_index("s") * NUM_CORES + lax.axis_index("c")
        pltpu.sync_copy(idx_hbm.at[wid], idx_v)

        def gather(i, b):
            return pltpu.make_async_copy(
                table_hbm.at[idx_v.at[i]], bufs.at[b], gsem.at[b]
            )

        gather(0, 0).start()

        n_k = dp // LANE
        cols = [jax.lax.iota(jnp.int32, LANE) + k * LANE for k in range(n_k)]
        masks = [c < d for c in cols]

        def step(l, b):
            gather(l, b).wait()

            @pl.when(l + 1 < n_l)
            def _():
                gather(l + 1, 1 - b).start()

            iota = jax.lax.iota(jnp.int32, LANE)
            bv = iota * 0 + b

            def tcols(c, carry2):
                cv = iota * 0 + c
                for jb in range(CHUNK // LANE):
                    jv = iota + jb * LANE
                    vec = plsc.load_gather(bufs, [bv, jv, cv])
                    buf_t[c, pl.ds(jb * LANE, LANE)] = vec
                return carry2

            lax.fori_loop(0, d, tcols, 0)
            pltpu.sync_copy(
                buf_t,
                out_hbm.at[pl.ds(l * d, d), pl.ds(wid * CHUNK, CHUNK)],
            )

        def body(g, carry):
            for b in range(2):
                step(g * 2 + b, b)
            return carry

        lax.fori_loop(0, n_l // 2, body, 0)

    return run(idx, table_pad)


def kernel(x, table):
    b, l = x.shape
    v, d = table.shape
    dp = (d + LANE - 1) // LANE * LANE
    table_pad = jnp.pad(table, ((0, 0), (0, dp - d)))
    idx = x.astype(jnp.int32).T.reshape(l, NW, CHUNK).transpose(1, 0, 2)
    out = _gather_t(idx, table_pad, v=v, d=d, dp=dp)
    return out.reshape(l, d, b).transpose(2, 0, 1)[:, None, :, :]


# speedup vs baseline: 1.3590x; 1.3590x over previous
import functools

import jax
import jax.numpy as jnp
from jax import lax
from jax.experimental import pallas as pl
from jax.experimental.pallas import tpu as pltpu
from jax.experimental.pallas import tpu_sc as plsc

NUM_CORES = 2
NUM_SUBCORES = 16
NW = NUM_CORES * NUM_SUBCORES

CHUNK = 128
LANE = 16


@functools.partial(jax.jit, static_argnames=("v", "dp"))
def _gather_rows(idx, table_pad, *, v, dp):
    nw, n_chunks, _ = idx.shape
    n = nw * n_chunks * CHUNK
    mesh = plsc.VectorSubcoreMesh(core_axis_name="c", subcore_axis_name="s")

    @functools.partial(
        pl.kernel,
        out_type=jax.ShapeDtypeStruct((n, dp), jnp.float32),
        mesh=mesh,
        scratch_types=[
            pltpu.VMEM((n_chunks, CHUNK), jnp.int32),
            pltpu.VMEM((2, CHUNK, dp), jnp.float32),
            pltpu.SemaphoreType.DMA((2,)),
            pltpu.SemaphoreType.DMA((2,)),
        ],
        compiler_params=pltpu.CompilerParams(use_tc_tiling_on_sc=False),
    )
    def run(idx_hbm, table_hbm, out_hbm, idx_v, bufs, gsem, wsem):
        wid = lax.axis_index("s") * NUM_CORES + lax.axis_index("c")
        base = wid * n_chunks * CHUNK
        pltpu.sync_copy(idx_hbm.at[wid], idx_v)

        def gather(i, b):
            return pltpu.make_async_copy(
                table_hbm.at[idx_v.at[i]], bufs.at[b], gsem.at[b]
            )

        def write(i, b):
            return pltpu.make_async_copy(
                bufs.at[b], out_hbm.at[pl.ds(base + i * CHUNK, CHUNK)], wsem.at[b]
            )

        for b in range(2):
            gather(b, b).start()

        def body(g, carry):
            for b in range(2):
                i = g * 2 + b
                gather(i, b).wait()
                write(i, b).start()
                nxt = i + 2

                @pl.when(nxt < n_chunks)
                def _():
                    write(i, b).wait()
                    gather(nxt, b).start()

            return carry

        lax.fori_loop(0, n_chunks // 2, body, 0)
        for b in range(2):
            write(0, b).wait()

    return run(idx, table_pad)


def kernel(x, table):
    b, l = x.shape
    v, d = table.shape
    n = b * l
    dp = (d + LANE - 1) // LANE * LANE
    idx = x.reshape(NW, n // (NW * CHUNK), CHUNK).astype(jnp.int32)
    table_pad = jnp.pad(table, ((0, 0), (0, dp - d)))
    out = _gather_rows(idx, table_pad, v=v, dp=dp)
    return out.reshape(b, l, dp)[:, None, :, :d]


# speedup vs baseline: 1.3595x; 1.0004x over previous
import functools

import jax
import jax.numpy as jnp
from jax import lax
from jax.experimental import pallas as pl
from jax.experimental.pallas import tpu as pltpu
from jax.experimental.pallas import tpu_sc as plsc

NUM_CORES = 2
NUM_SUBCORES = 16
NW = NUM_CORES * NUM_SUBCORES

CHUNK = 128
LANE = 16


@functools.partial(jax.jit, static_argnames=("v", "dp"))
def _gather_rows(idx, table_pad, *, v, dp):
    nw, n_chunks, _ = idx.shape
    n = nw * n_chunks * CHUNK
    mesh = plsc.VectorSubcoreMesh(core_axis_name="c", subcore_axis_name="s")

    @functools.partial(
        pl.kernel,
        out_type=jax.ShapeDtypeStruct((n, dp), jnp.float32),
        mesh=mesh,
        scratch_types=[
            pltpu.VMEM((n_chunks, CHUNK), jnp.int32),
            pltpu.VMEM((2, CHUNK, dp), jnp.float32),
            pltpu.SemaphoreType.DMA((2,)),
            pltpu.SemaphoreType.DMA((2,)),
        ],
        compiler_params=pltpu.CompilerParams(use_tc_tiling_on_sc=False),
    )
    def run(idx_hbm, table_hbm, out_hbm, idx_v, bufs, gsem, wsem):
        wid = lax.axis_index("s") * NUM_CORES + lax.axis_index("c")
        base = wid * n_chunks * CHUNK
        pltpu.sync_copy(idx_hbm.at[wid], idx_v)

        def gather(i, b):
            return pltpu.make_async_copy(
                table_hbm.at[idx_v.at[i]], bufs.at[b], gsem.at[b]
            )

        def write(i, b):
            return pltpu.make_async_copy(
                bufs.at[b], out_hbm.at[pl.ds(base + i * CHUNK, CHUNK)], wsem.at[b]
            )

        for b in range(2):
            gather(b, b).start()

        def body(g, carry):
            for b in range(2):
                i = g * 2 + b
                gather(i, b).wait()
                write(i, b).start()
                nxt = i + 2

                @pl.when(nxt < n_chunks)
                def _():
                    write(i, b).wait()
                    gather(nxt, b).start()

            return carry

        lax.fori_loop(0, n_chunks // 2, body, 0)
        for b in range(2):
            write(0, b).wait()

    return run(idx, table_pad)


def kernel(x, table):
    b, l = x.shape
    v, d = table.shape
    n = b * l
    dp = (d + LANE - 1) // LANE * LANE
    idx = x.reshape(NW, n // (NW * CHUNK), CHUNK).astype(jnp.int32)
    table_pad = jax.lax.dynamic_update_slice(jnp.zeros((v, dp), table.dtype), table, (0, 0))
    out = _gather_rows(idx, table_pad, v=v, dp=dp)
    return out.reshape(b, l, dp)[:, None, :, :d]
